# baseline (speedup 1.0000x reference)
import functools

import jax
import jax.numpy as jnp
from jax import lax
from jax.experimental import pallas as pl
from jax.experimental.pallas import tpu as pltpu
from jax.experimental.pallas import tpu_sc as plsc

D_E = 128
NUM_WORKERS = 32
NBUF = 8


def _sc_gather(idx2d, table, per_w, seq):
    n_batch = NUM_WORKERS * per_w
    n_chunks = per_w // 2
    chunk_idx = 2 * seq
    mesh = plsc.VectorSubcoreMesh(core_axis_name="c", subcore_axis_name="s")

    @functools.partial(
        pl.kernel,
        out_type=jax.ShapeDtypeStruct((n_batch, 56, D_E), jnp.float32),
        mesh=mesh,
        scratch_types=[
            pltpu.VMEM((n_chunks, 128), jnp.int32),
            pltpu.VMEM((NBUF, 112, D_E), jnp.float32),
        ]
        + [pltpu.SemaphoreType.DMA] * (2 * NBUF),
    )
    def k(idx_hbm, table_hbm, out_hbm, idx_v, rows_v, *sems):
        gs = sems[:NBUF]
        ws = sems[NBUF:]
        wid = lax.axis_index("s") * 2 + lax.axis_index("c")
        base_b = wid * per_w
        base_c = wid * n_chunks
        pltpu.sync_copy(idx_hbm.at[pl.ds(base_c, n_chunks)], idx_v)

        def gather(c, r):
            pltpu.async_copy(
                table_hbm.at[idx_v.at[c].at[pl.ds(0, chunk_idx)]],
                rows_v.at[r].at[pl.ds(0, chunk_idx)],
                gs[r],
            )

        def wait_gather(c, r):
            pltpu.make_async_copy(
                table_hbm.at[idx_v.at[c].at[pl.ds(0, chunk_idx)]],
                rows_v.at[r].at[pl.ds(0, chunk_idx)],
                gs[r],
            ).wait()

        def write(c, r):
            for h in range(2):
                pltpu.async_copy(
                    rows_v.at[r].at[pl.ds(h * seq, 56)],
                    out_hbm.at[base_b + 2 * c + h],
                    ws[r],
                )

        def wait_write(c, r):
            for h in range(2):
                pltpu.make_async_copy(
                    rows_v.at[r].at[pl.ds(h * seq, 56)],
                    out_hbm.at[base_b + 2 * c + h],
                    ws[r],
                ).wait()

        for r in range(NBUF):
            gather(r, r)

        def body(i, _):
            for r in range(NBUF):
                c = i * NBUF + r
                wait_gather(c, r)
                write(c, r)

                @pl.when(c + NBUF < n_chunks)
                def _():
                    wait_write(c, r)
                    gather(c + NBUF, r)

            return 0

        lax.fori_loop(0, n_chunks // NBUF, body, 0, unroll=False)

        for r in range(NBUF):
            wait_write(n_chunks - NBUF + r, r)

    return k(idx2d, table)


def kernel(input_ids, table):
    b, s = input_ids.shape
    per_w = b // NUM_WORKERS
    assert per_w * NUM_WORKERS == b and per_w % (2 * NBUF) == 0
    ids = input_ids.astype(jnp.int32).reshape(b // 2, 2 * s)
    ids = jnp.pad(ids, ((0, 0), (0, 128 - 2 * s)))
    out = _sc_gather(ids, table, per_w, s)
    return out[:, :s, :]


# speedup vs baseline: 1.1711x; 1.1711x over previous
import functools

import jax
import jax.numpy as jnp
from jax import lax
from jax.experimental import pallas as pl
from jax.experimental.pallas import tpu as pltpu
from jax.experimental.pallas import tpu_sc as plsc

D_E = 128
NUM_WORKERS = 32
NBUF = 8


def _sc_gather(idx2d, table, per_w, seq):
    n_batch = NUM_WORKERS * per_w
    n_chunks = per_w // 2
    chunk_idx = 2 * seq
    mesh = plsc.VectorSubcoreMesh(core_axis_name="c", subcore_axis_name="s")

    @functools.partial(
        pl.kernel,
        out_type=jax.ShapeDtypeStruct((n_batch, seq, D_E), jnp.float32),
        mesh=mesh,
        scratch_types=[
            pltpu.VMEM((n_chunks, 128), jnp.int32),
            pltpu.VMEM((NBUF, chunk_idx, D_E), jnp.float32),
        ]
        + [pltpu.SemaphoreType.DMA] * (2 * NBUF),
    )
    def k(idx_hbm, table_hbm, out_hbm, idx_v, rows_v, *sems):
        gs = sems[:NBUF]
        ws = sems[NBUF:]
        wid = lax.axis_index("s") * 2 + lax.axis_index("c")
        base_b = wid * per_w
        base_c = wid * n_chunks
        pltpu.sync_copy(idx_hbm.at[pl.ds(base_c, n_chunks)], idx_v)

        def gather(c, r):
            pltpu.async_copy(
                table_hbm.at[idx_v.at[c].at[pl.ds(0, chunk_idx)]],
                rows_v.at[r],
                gs[r],
            )

        def wait_gather(c, r):
            pltpu.make_async_copy(
                table_hbm.at[idx_v.at[c].at[pl.ds(0, chunk_idx)]],
                rows_v.at[r],
                gs[r],
            ).wait()

        def write(c, r):
            for h in range(2):
                pltpu.async_copy(
                    rows_v.at[r].at[pl.ds(h * seq, seq)],
                    out_hbm.at[base_b + 2 * c + h],
                    ws[r],
                )

        def wait_write(c, r):
            for h in range(2):
                pltpu.make_async_copy(
                    rows_v.at[r].at[pl.ds(h * seq, seq)],
                    out_hbm.at[base_b + 2 * c + h],
                    ws[r],
                ).wait()

        for r in range(NBUF):
            gather(r, r)

        LOOK = NBUF - 2

        def body(i, _):
            for r in range(NBUF):
                c = i * NBUF + r
                wait_gather(c, r)
                write(c, r)

                r2 = (r + LOOK) % NBUF
                c2 = c + LOOK

                @pl.when(jnp.logical_and(c2 >= NBUF, c2 < n_chunks))
                def _():
                    wait_write(c2 - NBUF, r2)
                    gather(c2, r2)

            return 0

        lax.fori_loop(0, n_chunks // NBUF, body, 0, unroll=False)

        for r in range(NBUF):
            wait_write(n_chunks - NBUF + r, r)

    return k(idx2d, table)


def kernel(input_ids, table):
    b, s = input_ids.shape
    per_w = b // NUM_WORKERS
    assert per_w * NUM_WORKERS == b and per_w % (2 * NBUF) == 0
    ids = input_ids.astype(jnp.int32).reshape(b // 2, 2 * s)
    ids = jnp.pad(ids, ((0, 0), (0, 128 - 2 * s)))
    return _sc_gather(ids, table, per_w, s)


# speedup vs baseline: 1.1740x; 1.0025x over previous
import functools

import jax
import jax.numpy as jnp
from jax import lax
from jax.experimental import pallas as pl
from jax.experimental.pallas import tpu as pltpu
from jax.experimental.pallas import tpu_sc as plsc

D_E = 128
NUM_WORKERS = 32
NBUF = 8


def _sc_gather(idx2d, table, per_w, seq):
    n_batch = NUM_WORKERS * per_w
    n_chunks = per_w // 2
    chunk_idx = 2 * seq
    mesh = plsc.VectorSubcoreMesh(core_axis_name="c", subcore_axis_name="s")

    @functools.partial(
        pl.kernel,
        out_type=jax.ShapeDtypeStruct((n_batch, seq, D_E), jnp.float32),
        mesh=mesh,
        scratch_types=[
            pltpu.VMEM((n_chunks, 128), jnp.int32),
            pltpu.VMEM((NBUF, chunk_idx, D_E), jnp.float32),
        ]
        + [pltpu.SemaphoreType.DMA] * (2 * NBUF),
    )
    def k(idx_hbm, table_hbm, out_hbm, idx_v, rows_v, *sems):
        gs = sems[:NBUF]
        ws = sems[NBUF:]
        wid = lax.axis_index("s") * 2 + lax.axis_index("c")
        base_b = wid * per_w
        base_c = wid * n_chunks
        pltpu.sync_copy(idx_hbm.at[pl.ds(base_c, n_chunks)], idx_v)

        def gather(c, r):
            pltpu.async_copy(
                table_hbm.at[idx_v.at[c].at[pl.ds(0, chunk_idx)]],
                rows_v.at[r],
                gs[r],
            )

        def wait_gather(c, r):
            pltpu.make_async_copy(
                table_hbm.at[idx_v.at[c].at[pl.ds(0, chunk_idx)]],
                rows_v.at[r],
                gs[r],
            ).wait()

        def write(c, r):
            for h in range(2):
                pltpu.async_copy(
                    rows_v.at[r].at[pl.ds(h * seq, seq)],
                    out_hbm.at[base_b + 2 * c + h],
                    ws[r],
                )

        def wait_write(c, r):
            for h in range(2):
                pltpu.make_async_copy(
                    rows_v.at[r].at[pl.ds(h * seq, seq)],
                    out_hbm.at[base_b + 2 * c + h],
                    ws[r],
                ).wait()

        for r in range(NBUF):
            gather(r, r)

        def body(i, _):
            for r in range(NBUF):
                c = i * NBUF + r
                wait_gather(c, r)
                write(c, r)

                @pl.when(c + NBUF < n_chunks)
                def _():
                    wait_write(c, r)
                    gather(c + NBUF, r)

            return 0

        lax.fori_loop(0, n_chunks // NBUF, body, 0, unroll=False)

        for r in range(NBUF):
            wait_write(n_chunks - NBUF + r, r)

    return k(idx2d, table)


def kernel(input_ids, table):
    b, s = input_ids.shape
    per_w = b // NUM_WORKERS
    assert per_w * NUM_WORKERS == b and per_w % (2 * NBUF) == 0
    ids = input_ids.astype(jnp.int32).reshape(b // 2, 2 * s)
    ids = jnp.pad(ids, ((0, 0), (0, 128 - 2 * s)))
    return _sc_gather(ids, table, per_w, s)
